# baseline (speedup 1.0000x reference)
import functools

import jax
import jax.numpy as jnp
from jax import lax
from jax.experimental import pallas as pl
from jax.experimental.pallas import tpu as pltpu
from jax.experimental.pallas import tpu_sc as plsc

N = 10000
D = 128
E = 320000
NP = 10240
NC = 2
NS = 16
L = 16
NW = NC * NS
EB = 128
NBE = 80
EPT = NBE * EB
EPAD = NW * EPT
NBUF = 4
RPT = NP // NW
RC = 80
SEG = NP // NS

_MESH = plsc.VectorSubcoreMesh(core_axis_name="c", subcore_axis_name="s")


def _wid():
    return lax.axis_index("s") * NC + lax.axis_index("c")


def _hist_body(dst_hbm, out_hbm, idx_v, ones_v, zer_v, hist_sh, sem):
    c = lax.axis_index("c")
    s = lax.axis_index("s")

    def fill_zero(i, _):
        zer_v[pl.ds(i * L, L)] = jnp.zeros((L,), jnp.float32)
        return 0

    lax.fori_loop(0, SEG // L, fill_zero, 0)
    for i in range(EB // L):
        ones_v[pl.ds(i * L, L)] = jnp.ones((L,), jnp.float32)
    pltpu.sync_copy(zer_v, hist_sh.at[pl.ds(s * SEG, SEG)])
    plsc.subcore_barrier()

    pltpu.sync_copy(dst_hbm.at[_wid()], idx_v)

    def step(j, _):
        pltpu.sync_copy(ones_v, hist_sh.at[idx_v.at[j]], add=True)
        return 0

    lax.fori_loop(0, NBE, step, 0)
    plsc.subcore_barrier()
    pltpu.sync_copy(hist_sh.at[pl.ds(s * SEG, SEG)],
                    out_hbm.at[c, pl.ds(s * SEG, SEG)])


_hist = functools.partial(
    pl.kernel,
    out_type=jax.ShapeDtypeStruct((NC, NP), jnp.float32),
    mesh=_MESH,
    scratch_types=[
        pltpu.VMEM((NBE, EB), jnp.int32),
        pltpu.VMEM((EB,), jnp.float32),
        pltpu.VMEM((SEG,), jnp.float32),
        pltpu.VMEM_SHARED((NP,), jnp.float32),
        pltpu.SemaphoreType.DMA,
    ],
)(_hist_body)


DH = D // 2


def _agg_body(ya_hbm, yb_hbm, src_hbm, dst_hbm, out_hbm, si_v, di_v,
              *scratch):
    rows = scratch[:NBUF]
    zr_v = scratch[NBUF]
    acc_sh = scratch[NBUF + 1]
    gsem = scratch[NBUF + 2:NBUF + 2 + NBUF]
    ssem = scratch[NBUF + 2 + NBUF:]
    c = lax.axis_index("c")
    s = lax.axis_index("s")

    pltpu.sync_copy(src_hbm.at[_wid()], si_v)
    pltpu.sync_copy(dst_hbm.at[_wid()], di_v)

    def zero_row(i, _):
        for cc in range(DH // L):
            zr_v[i, pl.ds(cc * L, L)] = jnp.zeros((L,), jnp.float32)
        return 0

    lax.fori_loop(0, EB, zero_row, 0)

    for h, y_hbm in enumerate((ya_hbm, yb_hbm)):
        for k in range(SEG // EB):
            pltpu.sync_copy(zr_v, acc_sh.at[pl.ds(s * SEG + k * EB, EB)])
        plsc.subcore_barrier()

        for b in range(NBUF):
            pltpu.async_copy(y_hbm.at[si_v.at[b]], rows[b], gsem[b])

        def group(g, _):
            base = g * NBUF
            for b in range(NBUF):
                j = base + b
                pltpu.make_async_copy(y_hbm.at[si_v.at[j]], rows[b],
                                      gsem[b]).wait()
                pltpu.async_copy(rows[b], acc_sh.at[di_v.at[j]], ssem[b],
                                 add=True)
            for b in range(NBUF):
                j = base + b
                pltpu.make_async_copy(rows[b], acc_sh.at[di_v.at[j]],
                                      ssem[b]).wait()
                pltpu.async_copy(y_hbm.at[si_v.at[j + NBUF]], rows[b],
                                 gsem[b])
            return 0

        lax.fori_loop(0, NBE // NBUF - 1, group, 0)
        for b in range(NBUF):
            j = NBE - NBUF + b
            pltpu.make_async_copy(y_hbm.at[si_v.at[j]], rows[b],
                                  gsem[b]).wait()
            pltpu.async_copy(rows[b], acc_sh.at[di_v.at[j]], ssem[b],
                             add=True)
        for b in range(NBUF):
            j = NBE - NBUF + b
            pltpu.make_async_copy(rows[b], acc_sh.at[di_v.at[j]],
                                  ssem[b]).wait()
        plsc.subcore_barrier()
        pltpu.sync_copy(acc_sh.at[pl.ds(s * SEG, SEG)],
                        out_hbm.at[c, h, pl.ds(s * SEG, SEG)])


_agg = functools.partial(
    pl.kernel,
    out_type=jax.ShapeDtypeStruct((NC, 2, NP, DH), jnp.float32),
    mesh=_MESH,
    compiler_params=pltpu.CompilerParams(use_tc_tiling_on_sc=False),
    scratch_types=(
        [
            pltpu.VMEM((NBE, EB), jnp.int32),
            pltpu.VMEM((NBE, EB), jnp.int32),
        ]
        + [pltpu.VMEM((EB, DH), jnp.float32) for _ in range(NBUF)]
        + [
            pltpu.VMEM((EB, DH), jnp.float32),
            pltpu.VMEM_SHARED((NP, DH), jnp.float32),
        ]
        + [pltpu.SemaphoreType.DMA for _ in range(2 * NBUF)]
    ),
)(_agg_body)


def _dinv_splat(dv_v, r):
    ridx = jnp.full((L,), r, jnp.int32)
    return plsc.load_gather(dv_v, [ridx])


def _scale_body(x_hbm, dinv_hbm, out_hbm, a_v, dv_v, sem):
    wid = _wid()
    pltpu.sync_copy(dinv_hbm.at[pl.ds(wid * RPT, RPT)], dv_v)

    def chunk(k, _):
        base = wid * RPT + k * RC
        pltpu.sync_copy(x_hbm.at[pl.ds(base, RC)], a_v)

        def row(r, _):
            dv = _dinv_splat(dv_v, k * RC + r)
            for cc in range(D // L):
                sl = pl.ds(cc * L, L)
                a_v[r, sl] = a_v[r, sl] * dv
            return 0

        lax.fori_loop(0, RC, row, 0)
        pltpu.sync_copy(a_v, out_hbm.at[pl.ds(base, RC)])
        return 0

    lax.fori_loop(0, RPT // RC, chunk, 0)


_scale = functools.partial(
    pl.kernel,
    out_type=jax.ShapeDtypeStruct((NP, D), jnp.float32),
    mesh=_MESH,
    compiler_params=pltpu.CompilerParams(needs_layout_passes=False),
    scratch_types=[
        pltpu.VMEM((RC, D), jnp.float32),
        pltpu.VMEM((RPT,), jnp.float32),
        pltpu.SemaphoreType.DMA,
    ],
)(_scale_body)


def _make_combine(relu):
    def body(p_hbm, y_hbm, dinv_hbm, b_hbm, out_hbm, a_v, c_v, t_v, y_v, dv_v,
             bias_v, sem):
        wid = _wid()
        pltpu.sync_copy(dinv_hbm.at[pl.ds(wid * RPT, RPT)], dv_v)
        pltpu.sync_copy(b_hbm, bias_v)

        def chunk(k, _):
            base = wid * RPT + k * RC
            pltpu.sync_copy(p_hbm.at[0, 0, pl.ds(base, RC)], a_v)
            pltpu.sync_copy(p_hbm.at[1, 0, pl.ds(base, RC)], t_v)

            def row_add_a(r, _):
                for cc in range(DH // L):
                    sl = pl.ds(cc * L, L)
                    a_v[r, sl] = a_v[r, sl] + t_v[r, sl]
                return 0

            lax.fori_loop(0, RC, row_add_a, 0)
            pltpu.sync_copy(p_hbm.at[0, 1, pl.ds(base, RC)], c_v)
            pltpu.sync_copy(p_hbm.at[1, 1, pl.ds(base, RC)], t_v)

            def row_add_c(r, _):
                for cc in range(DH // L):
                    sl = pl.ds(cc * L, L)
                    c_v[r, sl] = c_v[r, sl] + t_v[r, sl]
                return 0

            lax.fori_loop(0, RC, row_add_c, 0)
            pltpu.sync_copy(y_hbm.at[pl.ds(base, RC)], y_v)

            def row_fin(r, _):
                dv = _dinv_splat(dv_v, k * RC + r)
                for cc in range(D // L):
                    sl = pl.ds(cc * L, L)
                    hsl = pl.ds((cc % (DH // L)) * L, L)
                    p = a_v[r, hsl] if cc < DH // L else c_v[r, hsl]
                    t = (p + y_v[r, sl]) * dv + bias_v[sl]
                    if relu:
                        t = jnp.maximum(t, 0.0) * dv
                    y_v[r, sl] = t
                return 0

            lax.fori_loop(0, RC, row_fin, 0)
            pltpu.sync_copy(y_v, out_hbm.at[pl.ds(base, RC)])
            return 0

        lax.fori_loop(0, RPT // RC, chunk, 0)

    return functools.partial(
        pl.kernel,
        out_type=jax.ShapeDtypeStruct((NP, D), jnp.float32),
        mesh=_MESH,
        compiler_params=pltpu.CompilerParams(needs_layout_passes=False),
        scratch_types=[
            pltpu.VMEM((RC, DH), jnp.float32),
            pltpu.VMEM((RC, DH), jnp.float32),
            pltpu.VMEM((RC, DH), jnp.float32),
            pltpu.VMEM((RC, D), jnp.float32),
            pltpu.VMEM((RPT,), jnp.float32),
            pltpu.VMEM((D,), jnp.float32),
            pltpu.SemaphoreType.DMA,
        ],
    )(body)


_combine_relu = _make_combine(True)
_combine_plain = _make_combine(False)


def _dinv_body(h_ref, o_ref):
    deg = h_ref[0] + h_ref[1] + 1.0
    o_ref[...] = lax.rsqrt(deg)


def _dinv(hist):
    out = pl.pallas_call(
        _dinv_body,
        out_shape=jax.ShapeDtypeStruct((NP // D, D), jnp.float32),
    )(hist.reshape(NC, NP // D, D))
    return out.reshape(NP)


def _mm_body(x_ref, w_ref, o_ref):
    o_ref[...] = lax.dot_general(
        x_ref[...], w_ref[...], (((1,), (1,)), ((), ())),
        preferred_element_type=jnp.float32,
        precision=lax.Precision.HIGHEST)


def _mm(x, w):
    blk = 256
    return pl.pallas_call(
        _mm_body,
        grid=(NP // blk,),
        in_specs=[
            pl.BlockSpec((blk, D), lambda i: (i, 0)),
            pl.BlockSpec((D, D), lambda i: (0, 0)),
        ],
        out_specs=pl.BlockSpec((blk, D), lambda i: (i, 0)),
        out_shape=jax.ShapeDtypeStruct((NP, D), jnp.float32),
    )(x, w)


def kernel(x, edge_index, W1, b1, W2, b2):
    src = edge_index[0].astype(jnp.int32)
    dst = edge_index[1].astype(jnp.int32)
    pad = jnp.full((EPAD - E,), N, jnp.int32)
    src3 = jnp.concatenate([src, pad]).reshape(NW, NBE, EB)
    dst3 = jnp.concatenate([dst, pad]).reshape(NW, NBE, EB)
    x_pad = jnp.zeros((NP, D), jnp.float32).at[:N].set(x)

    hist = _hist(dst3)
    xw1 = _mm(x_pad, W1)
    dinv = _dinv(hist)
    y1 = _scale(xw1, dinv)
    p1 = _agg(y1[:, :DH], y1[:, DH:], src3, dst3)
    h = _combine_relu(p1, y1, dinv, b1)
    y2 = _mm(h, W2)
    p2 = _agg(y2[:, :DH], y2[:, DH:], src3, dst3)
    out = _combine_plain(p2, y2, dinv, b2)
    return out[:N]


# speedup vs baseline: 2.6402x; 2.6402x over previous
import functools

import jax
import jax.numpy as jnp
from jax import lax
from jax.experimental import pallas as pl
from jax.experimental.pallas import tpu as pltpu
from jax.experimental.pallas import tpu_sc as plsc

N = 10000
D = 128
E = 320000
NP = 10240
NC = 2
NS = 16
L = 16
NW = NC * NS
EB = 128
NBE = 80
EPT = NBE * EB
EPAD = NW * EPT
NBUF = 4
RPT = NP // NW
RC = 80
SEG = NP // NS

_MESH = plsc.VectorSubcoreMesh(core_axis_name="c", subcore_axis_name="s")


def _wid():
    return lax.axis_index("s") * NC + lax.axis_index("c")


def _hist_body(dst_hbm, out_hbm, idx_v, ones_v, zer_v, hist_sh, sem):
    c = lax.axis_index("c")
    s = lax.axis_index("s")

    def fill_zero(i, _):
        zer_v[pl.ds(i * L, L)] = jnp.zeros((L,), jnp.float32)
        return 0

    lax.fori_loop(0, SEG // L, fill_zero, 0)
    for i in range(EB // L):
        ones_v[pl.ds(i * L, L)] = jnp.ones((L,), jnp.float32)
    pltpu.sync_copy(zer_v, hist_sh.at[pl.ds(s * SEG, SEG)])
    plsc.subcore_barrier()

    pltpu.sync_copy(dst_hbm.at[_wid()], idx_v)

    def step(j, _):
        pltpu.sync_copy(ones_v, hist_sh.at[idx_v.at[j]], add=True)
        return 0

    lax.fori_loop(0, NBE, step, 0)
    plsc.subcore_barrier()
    pltpu.sync_copy(hist_sh.at[pl.ds(s * SEG, SEG)],
                    out_hbm.at[c, pl.ds(s * SEG, SEG)])


_hist = functools.partial(
    pl.kernel,
    out_type=jax.ShapeDtypeStruct((NC, NP), jnp.float32),
    mesh=_MESH,
    scratch_types=[
        pltpu.VMEM((NBE, EB), jnp.int32),
        pltpu.VMEM((EB,), jnp.float32),
        pltpu.VMEM((SEG,), jnp.float32),
        pltpu.VMEM_SHARED((NP,), jnp.float32),
        pltpu.SemaphoreType.DMA,
    ],
)(_hist_body)


DH = D // 2


def _agg_body(ya_hbm, yb_hbm, src_hbm, dst_hbm, out_hbm, si_v, di_v,
              *scratch):
    rows = scratch[:NBUF]
    zr_v = scratch[NBUF]
    acc_sh = scratch[NBUF + 1]
    gsem = scratch[NBUF + 2:NBUF + 2 + NBUF]
    ssem = scratch[NBUF + 2 + NBUF:]
    c = lax.axis_index("c")
    s = lax.axis_index("s")

    pltpu.sync_copy(src_hbm.at[_wid()], si_v)
    pltpu.sync_copy(dst_hbm.at[_wid()], di_v)

    def zero_row(i, _):
        for cc in range(DH // L):
            zr_v[i, pl.ds(cc * L, L)] = jnp.zeros((L,), jnp.float32)
        return 0

    lax.fori_loop(0, EB, zero_row, 0)

    for h, y_hbm in enumerate((ya_hbm, yb_hbm)):
        for k in range(SEG // EB):
            pltpu.sync_copy(zr_v, acc_sh.at[pl.ds(s * SEG + k * EB, EB)])
        plsc.subcore_barrier()

        for b in range(NBUF):
            pltpu.async_copy(y_hbm.at[si_v.at[b]], rows[b], gsem[b])

        def group(g, _):
            base = g * NBUF
            for b in range(NBUF):
                j = base + b
                pltpu.make_async_copy(y_hbm.at[si_v.at[j]], rows[b],
                                      gsem[b]).wait()
                pltpu.async_copy(rows[b], acc_sh.at[di_v.at[j]], ssem[b],
                                 add=True)
            for b in range(NBUF):
                j = base + b
                pltpu.make_async_copy(rows[b], acc_sh.at[di_v.at[j]],
                                      ssem[b]).wait()
                pltpu.async_copy(y_hbm.at[si_v.at[j + NBUF]], rows[b],
                                 gsem[b])
            return 0

        lax.fori_loop(0, NBE // NBUF - 1, group, 0)
        for b in range(NBUF):
            j = NBE - NBUF + b
            pltpu.make_async_copy(y_hbm.at[si_v.at[j]], rows[b],
                                  gsem[b]).wait()
            pltpu.async_copy(rows[b], acc_sh.at[di_v.at[j]], ssem[b],
                             add=True)
        for b in range(NBUF):
            j = NBE - NBUF + b
            pltpu.make_async_copy(rows[b], acc_sh.at[di_v.at[j]],
                                  ssem[b]).wait()
        plsc.subcore_barrier()
        pltpu.sync_copy(acc_sh.at[pl.ds(s * SEG, SEG)],
                        out_hbm.at[c, h, pl.ds(s * SEG, SEG)])


_agg = functools.partial(
    pl.kernel,
    out_type=jax.ShapeDtypeStruct((NC, 2, NP, DH), jnp.float32),
    mesh=_MESH,
    compiler_params=pltpu.CompilerParams(use_tc_tiling_on_sc=False),
    scratch_types=(
        [
            pltpu.VMEM((NBE, EB), jnp.int32),
            pltpu.VMEM((NBE, EB), jnp.int32),
        ]
        + [pltpu.VMEM((EB, DH), jnp.float32) for _ in range(NBUF)]
        + [
            pltpu.VMEM((EB, DH), jnp.float32),
            pltpu.VMEM_SHARED((NP, DH), jnp.float32),
        ]
        + [pltpu.SemaphoreType.DMA for _ in range(2 * NBUF)]
    ),
)(_agg_body)


def _dinv_splat(dv_v, r):
    ridx = jnp.full((L,), r, jnp.int32)
    return plsc.load_gather(dv_v, [ridx])


def _scale_body(x_hbm, dinv_hbm, out_hbm, a_v, dv_v, sem):
    wid = _wid()
    pltpu.sync_copy(dinv_hbm.at[pl.ds(wid * RPT, RPT)], dv_v)

    def chunk(k, _):
        base = wid * RPT + k * RC
        pltpu.sync_copy(x_hbm.at[pl.ds(base, RC)], a_v)

        def row(r, _):
            dv = _dinv_splat(dv_v, k * RC + r)
            for cc in range(D // L):
                sl = pl.ds(cc * L, L)
                a_v[r, sl] = a_v[r, sl] * dv
            return 0

        lax.fori_loop(0, RC, row, 0)
        pltpu.sync_copy(a_v, out_hbm.at[pl.ds(base, RC)])
        return 0

    lax.fori_loop(0, RPT // RC, chunk, 0)


_scale = functools.partial(
    pl.kernel,
    out_type=jax.ShapeDtypeStruct((NP, D), jnp.float32),
    mesh=_MESH,
    compiler_params=pltpu.CompilerParams(needs_layout_passes=False),
    scratch_types=[
        pltpu.VMEM((RC, D), jnp.float32),
        pltpu.VMEM((RPT,), jnp.float32),
        pltpu.SemaphoreType.DMA,
    ],
)(_scale_body)


def _make_combine(relu):
    def body(p_hbm, y_hbm, dinv_hbm, b_hbm, out_hbm, a_v, c_v, t_v, y_v, dv_v,
             bias_v, sem):
        wid = _wid()
        pltpu.sync_copy(dinv_hbm.at[pl.ds(wid * RPT, RPT)], dv_v)
        pltpu.sync_copy(b_hbm, bias_v)

        def chunk(k, _):
            base = wid * RPT + k * RC
            pltpu.sync_copy(p_hbm.at[0, 0, pl.ds(base, RC)], a_v)
            pltpu.sync_copy(p_hbm.at[1, 0, pl.ds(base, RC)], t_v)

            def row_add_a(r, _):
                for cc in range(DH // L):
                    sl = pl.ds(cc * L, L)
                    a_v[r, sl] = a_v[r, sl] + t_v[r, sl]
                return 0

            lax.fori_loop(0, RC, row_add_a, 0)
            pltpu.sync_copy(p_hbm.at[0, 1, pl.ds(base, RC)], c_v)
            pltpu.sync_copy(p_hbm.at[1, 1, pl.ds(base, RC)], t_v)

            def row_add_c(r, _):
                for cc in range(DH // L):
                    sl = pl.ds(cc * L, L)
                    c_v[r, sl] = c_v[r, sl] + t_v[r, sl]
                return 0

            lax.fori_loop(0, RC, row_add_c, 0)
            pltpu.sync_copy(y_hbm.at[pl.ds(base, RC)], y_v)

            def row_fin(r, _):
                dv = _dinv_splat(dv_v, k * RC + r)
                for cc in range(D // L):
                    sl = pl.ds(cc * L, L)
                    hsl = pl.ds((cc % (DH // L)) * L, L)
                    p = a_v[r, hsl] if cc < DH // L else c_v[r, hsl]
                    t = (p + y_v[r, sl]) * dv + bias_v[sl]
                    if relu:
                        t = jnp.maximum(t, 0.0) * dv
                    y_v[r, sl] = t
                return 0

            lax.fori_loop(0, RC, row_fin, 0)
            pltpu.sync_copy(y_v, out_hbm.at[pl.ds(base, RC)])
            return 0

        lax.fori_loop(0, RPT // RC, chunk, 0)

    return functools.partial(
        pl.kernel,
        out_type=jax.ShapeDtypeStruct((NP, D), jnp.float32),
        mesh=_MESH,
        compiler_params=pltpu.CompilerParams(needs_layout_passes=False),
        scratch_types=[
            pltpu.VMEM((RC, DH), jnp.float32),
            pltpu.VMEM((RC, DH), jnp.float32),
            pltpu.VMEM((RC, DH), jnp.float32),
            pltpu.VMEM((RC, D), jnp.float32),
            pltpu.VMEM((RPT,), jnp.float32),
            pltpu.VMEM((D,), jnp.float32),
            pltpu.SemaphoreType.DMA,
        ],
    )(body)


_combine_relu = _make_combine(True)
_combine_plain = _make_combine(False)


def _dinv_body(h_ref, o_ref):
    deg = h_ref[0] + h_ref[1] + 1.0
    o_ref[...] = lax.rsqrt(deg)


def _dinv(hist):
    out = pl.pallas_call(
        _dinv_body,
        out_shape=jax.ShapeDtypeStruct((NP // D, D), jnp.float32),
    )(hist.reshape(NC, NP // D, D))
    return out.reshape(NP)


def _mm_body(x_ref, w_ref, o_ref):
    o_ref[...] = lax.dot_general(
        x_ref[...], w_ref[...], (((1,), (1,)), ((), ())),
        preferred_element_type=jnp.float32,
        precision=lax.Precision.HIGHEST)


def _mm(x, w):
    blk = 256
    return pl.pallas_call(
        _mm_body,
        grid=(NP // blk,),
        in_specs=[
            pl.BlockSpec((blk, D), lambda i: (i, 0)),
            pl.BlockSpec((D, D), lambda i: (0, 0)),
        ],
        out_specs=pl.BlockSpec((blk, D), lambda i: (i, 0)),
        out_shape=jax.ShapeDtypeStruct((NP, D), jnp.float32),
    )(x, w)


def kernel(x, edge_index, W1, b1, W2, b2):
    src = edge_index[0].astype(jnp.int32)
    dst = edge_index[1].astype(jnp.int32)
    pad = N + jnp.arange(EPAD - E, dtype=jnp.int32) % (NP - N)
    src3 = jnp.concatenate([src, pad]).reshape(NW, NBE, EB)
    dst3 = jnp.concatenate([dst, pad]).reshape(NW, NBE, EB)
    x_pad = jnp.zeros((NP, D), jnp.float32).at[:N].set(x)

    hist = _hist(dst3)
    xw1 = _mm(x_pad, W1)
    dinv = _dinv(hist)
    y1 = _scale(xw1, dinv)
    p1 = _agg(y1[:, :DH], y1[:, DH:], src3, dst3)
    h = _combine_relu(p1, y1, dinv, b1)
    y2 = _mm(h, W2)
    p2 = _agg(y2[:, :DH], y2[:, DH:], src3, dst3)
    out = _combine_plain(p2, y2, dinv, b2)
    return out[:N]
